# baseline (speedup 1.0000x reference)
from functools import partial

import jax
import jax.numpy as jnp
from jax.experimental import pallas as pl
from jax.experimental.pallas import tpu as pltpu

_EPS = 1e-07
_DD = 2


def _partial_kernel(x_ref, t_ref, inter_ref, card_ref, *, c, dd):
    k = pl.program_id(1)

    @pl.when(k == 0)
    def _():
        inter_ref[...] = jnp.zeros_like(inter_ref)
        card_ref[...] = jnp.zeros_like(card_ref)

    x = x_ref[...]
    t = t_ref[...]
    b = x.shape[0]
    prod = (x * t).reshape(b, c * dd, 64, 64)
    card = (x + t).reshape(b, c * dd, 64, 64)
    inter_ref[...] += jnp.sum(prod, axis=1)[None]
    card_ref[...] += jnp.sum(card, axis=1)[None]


def kernel(x, target):
    b, c, d, h, w = x.shape
    dd = _DD
    kb = d // dd
    kpp = kb // 2

    in_spec = pl.BlockSpec((b, c, dd, h, w),
                           lambda p, k: (0, 0, p * kpp + k, 0, 0))
    out_spec = pl.BlockSpec((1, b, h, w), lambda p, k: (p, 0, 0, 0))

    inter_p, card_p = pl.pallas_call(
        partial(_partial_kernel, c=c, dd=dd),
        out_shape=(jax.ShapeDtypeStruct((2, b, h, w), jnp.float32),
                   jax.ShapeDtypeStruct((2, b, h, w), jnp.float32)),
        grid=(2, kpp),
        in_specs=[in_spec, in_spec],
        out_specs=(out_spec, out_spec),
        compiler_params=pltpu.CompilerParams(
            dimension_semantics=("parallel", "arbitrary"),
            vmem_limit_bytes=52 * 1024 * 1024,
        ),
    )(x, target)

    inter = jnp.sum(inter_p.reshape(2, b, -1), axis=(0, 2))
    card = jnp.sum(card_p.reshape(2, b, -1), axis=(0, 2))
    dice = 1.0 - 2.0 * inter / (card + _EPS)
    max_val = jnp.max(dice)
    weights = dice / max_val
    return jnp.mean(max_val * weights)


# speedup vs baseline: 1.2053x; 1.2053x over previous
from functools import partial

import jax
import jax.numpy as jnp
from jax.experimental import pallas as pl
from jax.experimental.pallas import tpu as pltpu

_EPS = 1e-07
_DD = 4


def _dice_kernel(x_ref, t_ref, o_ref, acc_i, acc_c, *, c, dd, kb):
    k = pl.program_id(0)

    @pl.when(k == 0)
    def _():
        acc_i[...] = jnp.zeros_like(acc_i)
        acc_c[...] = jnp.zeros_like(acc_c)

    x = x_ref[...]
    t = t_ref[...]
    b = x.shape[0]
    h, w = x.shape[3], x.shape[4]
    prod = (x * t).reshape(b, c * dd, h, w)
    card = (x + t).reshape(b, c * dd, h, w)
    acc_i[...] += jnp.sum(prod, axis=1)
    acc_c[...] += jnp.sum(card, axis=1)

    @pl.when(k == kb - 1)
    def _():
        inter = jnp.sum(jnp.sum(acc_i[...], axis=2), axis=1, keepdims=True)
        card_s = jnp.sum(jnp.sum(acc_c[...], axis=2), axis=1, keepdims=True)
        dice = 1.0 - 2.0 * inter / (card_s + _EPS)
        max_val = jnp.max(dice)
        weights = dice / max_val
        loss = jnp.mean(max_val * weights)
        o_ref[...] = jnp.full(o_ref.shape, loss, jnp.float32)


def kernel(x, target):
    b, c, d, h, w = x.shape
    dd = _DD
    kb = d // dd

    in_spec = pl.BlockSpec((b, c, dd, h, w), lambda k: (0, 0, k, 0, 0))
    out_spec = pl.BlockSpec((8, 128), lambda k: (0, 0))

    out = pl.pallas_call(
        partial(_dice_kernel, c=c, dd=dd, kb=kb),
        out_shape=jax.ShapeDtypeStruct((8, 128), jnp.float32),
        grid=(kb,),
        in_specs=[in_spec, in_spec],
        out_specs=out_spec,
        scratch_shapes=[pltpu.VMEM((b, h, w), jnp.float32),
                        pltpu.VMEM((b, h, w), jnp.float32)],
        compiler_params=pltpu.CompilerParams(
            vmem_limit_bytes=52 * 1024 * 1024,
        ),
    )(x, target)

    return out[0, 0]
